# baseline (speedup 1.0000x reference)
import jax
import jax.numpy as jnp
from jax import lax
from jax.experimental import pallas as pl
from jax.experimental.pallas import tpu as pltpu
from jax.experimental.pallas import tpu_sc as plsc

NUM_CLASSES = 1000000
EMBED_DIM = 32
BATCH = 16384
FIELDS = 26

_NC, _NS = 2, 16
_NW = _NC * _NS
_BW = BATCH // _NW
_NBUF = 4


def _embed_body(xt_hbm, table_hbm, out_hbm, idx_v, rows, sem_i, sems_g, sems_o):
    wid = lax.axis_index("s") * _NC + lax.axis_index("c")
    b0 = wid * _BW
    idx_cps = [
        pltpu.async_copy(xt_hbm.at[f, pl.ds(b0, _BW)], idx_v.at[f], sem_i)
        for f in range(FIELDS)
    ]
    for cp in idx_cps:
        cp.wait()

    lag = _NBUF - 1
    g_cps = [None] * _NBUF
    out_cps = [None] * _NBUF
    for j in range(FIELDS + lag):
        if j < FIELDS:
            b = j % _NBUF
            if out_cps[b] is not None:
                out_cps[b].wait()
            g_cps[b] = pltpu.async_copy(
                table_hbm.at[idx_v.at[j]], rows[b], sems_g[b])
        if j >= lag:
            i = j - lag
            b = i % _NBUF
            g_cps[b].wait()
            out_cps[b] = pltpu.async_copy(
                rows[b], out_hbm.at[i, pl.ds(b0, _BW), :], sems_o[b])
    for cp in out_cps:
        if cp is not None:
            cp.wait()


def kernel(x, table):
    mesh = plsc.VectorSubcoreMesh(core_axis_name="c", subcore_axis_name="s",
                                  num_cores=_NC, num_subcores=_NS)
    xt = x.T
    out = pl.kernel(
        _embed_body,
        out_type=jax.ShapeDtypeStruct((FIELDS, BATCH, EMBED_DIM), jnp.float32),
        mesh=mesh,
        scratch_types=[
            pltpu.VMEM((FIELDS, _BW), jnp.int32),
            [pltpu.VMEM((_BW, EMBED_DIM), jnp.float32)] * _NBUF,
            pltpu.SemaphoreType.DMA,
            [pltpu.SemaphoreType.DMA] * _NBUF,
            [pltpu.SemaphoreType.DMA] * _NBUF,
        ],
        compiler_params=pltpu.CompilerParams(use_tc_tiling_on_sc=False),
    )(xt, table)
    return out.transpose(1, 0, 2)
